# baseline (speedup 1.0000x reference)
import functools

import jax
import jax.numpy as jnp
from jax import lax
from jax.experimental import pallas as pl
from jax.experimental.pallas import tpu as pltpu
from jax.experimental.pallas import tpu_sc as plsc

NC = 2
NS = 16
LANES = 16
CHUNK = 64
STRIP = 256
NSTRIP = 4
NGROUP = 8
NSPAN = 8
VPS = STRIP // LANES


def _span_sum_body(S, seq_hbm, pos_hbm, part_hbm,
                   pos_v, bufP0, bufP1, bufA, bufB, outv0, outv1,
                   semP0, semP1, semA, semB, semO0, semO1):
    wid = lax.axis_index("s") * NC + lax.axis_index("c")
    g = wid // NSTRIP
    c0 = (wid % NSTRIP) * STRIP

    pltpu.sync_copy(pos_hbm, pos_v)
    pv = pos_v[...]

    los, his, a0s, ms, ns = [], [], [], [], []
    for s in range(NSPAN):
        b, e = s // 2, s % 2
        s0 = jnp.clip(pv[4 * b + 2 * e], 0, S - 1)
        e0 = jnp.maximum(s0, jnp.minimum(pv[4 * b + 2 * e + 1], S - 1))
        n = e0 - s0 + 1
        q = (n + NGROUP - 1) // NGROUP
        lo = jnp.minimum(s0 + g * q, e0 + 1)
        hi = jnp.minimum(e0 + 1, lo + q)
        a0 = (lo // 8) * 8
        los.append(lo)
        his.append(hi)
        a0s.append(a0)
        ms.append(jnp.where(lo < hi, (hi - a0 + CHUNK - 1) // CHUNK, 0))
        ns.append(n)

    def dma_r0(s, k):
        return jnp.minimum(a0s[s] + k * CHUNK, S - CHUNK)

    def src(s, k):
        return seq_hbm.at[s // 2, pl.ds(dma_r0(s, k), CHUNK),
                          pl.ds(c0, STRIP)]

    def issue(s, k, buf, sem):
        @pl.when(k < ms[s])
        def _():
            pltpu.async_copy(src(s, k), buf, sem)

    def drain(s, k, buf, sem):
        @pl.when(k < ms[s])
        def _():
            pltpu.make_async_copy(src(s, k), buf, sem).wait()

    def acc_chunk(s, k, buf, acc):
        r0 = a0s[s] + k * CHUNK
        base = dma_r0(s, k)
        jlo = jnp.maximum(los[s], r0) - base
        jhi = jnp.minimum(his[s], r0 + CHUNK) - base

        def row_body(j, acc):
            return tuple(acc[h] + buf[j, pl.ds(h * LANES, LANES)]
                         for h in range(VPS))

        return lax.fori_loop(jlo, jhi, row_body, acc)

    def out_ref(s):
        return part_hbm.at[pl.ds((g * NSPAN + s) * 1024 + c0, STRIP)]

    zeros = tuple(jnp.zeros((LANES,), jnp.float32) for _ in range(VPS))
    issue(0, 0, bufP0, semP0)
    for s in range(NSPAN):
        bufP, semP = (bufP0, semP0) if s % 2 == 0 else (bufP1, semP1)
        issue(s, 1, bufA, semA)
        if s + 1 < NSPAN:
            nbufP, nsemP = (bufP1, semP1) if s % 2 == 0 else (bufP0, semP0)
            issue(s + 1, 0, nbufP, nsemP)
        drain(s, 0, bufP, semP)
        acc = acc_chunk(s, 0, bufP, zeros)

        def make_pair(s):
            def pair_body(k2, acc):
                a = 1 + 2 * k2
                issue(s, a + 1, bufB, semB)
                drain(s, a, bufA, semA)
                acc = acc_chunk(s, a, bufA, acc)
                issue(s, a + 2, bufA, semA)
                drain(s, a + 1, bufB, semB)
                return acc_chunk(s, a + 1, bufB, acc)
            return pair_body

        acc = lax.fori_loop(0, ms[s] // 2, make_pair(s), acc)

        ov, semO = (outv0, semO0) if s % 2 == 0 else (outv1, semO1)
        if s >= 2:
            pltpu.make_async_copy(ov, out_ref(s - 2), semO).wait()
        nv = jnp.full((LANES,), ns[s], jnp.int32).astype(jnp.float32)
        for h in range(VPS):
            ov[pl.ds(h * LANES, LANES)] = acc[h] / nv
        pltpu.async_copy(ov, out_ref(s), semO)

    pltpu.make_async_copy(outv0, out_ref(NSPAN - 2), semO0).wait()
    pltpu.make_async_copy(outv1, out_ref(NSPAN - 1), semO1).wait()


def kernel(sequence_output, entity_positions):
    B, S, H = sequence_output.shape
    pos16 = entity_positions.reshape(B * 4).astype(jnp.int32)

    mesh = plsc.VectorSubcoreMesh(
        core_axis_name="c", subcore_axis_name="s",
        num_cores=NC, num_subcores=NS)
    fn = pl.kernel(
        functools.partial(_span_sum_body, S),
        out_type=jax.ShapeDtypeStruct((NGROUP * NSPAN * H,), jnp.float32),
        mesh=mesh,
        compiler_params=pltpu.CompilerParams(needs_layout_passes=False),
        scratch_types=[
            pltpu.VMEM((16,), jnp.int32),
            pltpu.VMEM((CHUNK, STRIP), jnp.float32),
            pltpu.VMEM((CHUNK, STRIP), jnp.float32),
            pltpu.VMEM((CHUNK, STRIP), jnp.float32),
            pltpu.VMEM((CHUNK, STRIP), jnp.float32),
            pltpu.VMEM((STRIP,), jnp.float32),
            pltpu.VMEM((STRIP,), jnp.float32),
            pltpu.SemaphoreType.DMA,
            pltpu.SemaphoreType.DMA,
            pltpu.SemaphoreType.DMA,
            pltpu.SemaphoreType.DMA,
            pltpu.SemaphoreType.DMA,
            pltpu.SemaphoreType.DMA,
        ],
    )
    partials = fn(sequence_output, pos16)
    means = partials.reshape(NGROUP, NSPAN, H).sum(axis=0)
    return means[0::2], means[1::2]


# speedup vs baseline: 1.0259x; 1.0259x over previous
import functools

import jax
import jax.numpy as jnp
from jax import lax
from jax.experimental import pallas as pl
from jax.experimental.pallas import tpu as pltpu
from jax.experimental.pallas import tpu_sc as plsc

NC = 2
NS = 16
LANES = 16
CHUNK = 32
STRIP = 256
NSTRIP = 4
NGROUP = 8
NSPAN = 8
VPS = STRIP // LANES


def _span_sum_body(S, seq_hbm, pos_hbm, part_hbm,
                   pos_v, bufP0, bufP1, bufA, bufB, bufC, outv0, outv1,
                   semP0, semP1, semA, semB, semC, semO0, semO1):
    wid = lax.axis_index("s") * NC + lax.axis_index("c")
    g = wid // NSTRIP
    c0 = (wid % NSTRIP) * STRIP

    pltpu.sync_copy(pos_hbm, pos_v)
    pv = pos_v[...]

    los, his, a0s, ms, ns = [], [], [], [], []
    for s in range(NSPAN):
        b, e = s // 2, s % 2
        s0 = jnp.clip(pv[4 * b + 2 * e], 0, S - 1)
        e0 = jnp.maximum(s0, jnp.minimum(pv[4 * b + 2 * e + 1], S - 1))
        n = e0 - s0 + 1
        q = (n + NGROUP - 1) // NGROUP
        lo = jnp.minimum(s0 + g * q, e0 + 1)
        hi = jnp.minimum(e0 + 1, lo + q)
        a0 = (lo // 8) * 8
        los.append(lo)
        his.append(hi)
        a0s.append(a0)
        ms.append(jnp.where(lo < hi, (hi - a0 + CHUNK - 1) // CHUNK, 0))
        ns.append(n)

    def dma_r0(s, k):
        return jnp.minimum(a0s[s] + k * CHUNK, S - CHUNK)

    def src(s, k):
        return seq_hbm.at[s // 2, pl.ds(dma_r0(s, k), CHUNK),
                          pl.ds(c0, STRIP)]

    def issue(s, k, buf, sem):
        @pl.when(k < ms[s])
        def _():
            pltpu.async_copy(src(s, k), buf, sem)

    def drain(s, k, buf, sem):
        @pl.when(k < ms[s])
        def _():
            pltpu.make_async_copy(src(s, k), buf, sem).wait()

    def acc_chunk(s, k, buf, acc):
        r0 = a0s[s] + k * CHUNK
        base = dma_r0(s, k)
        jlo = jnp.maximum(los[s], r0) - base
        jhi = jnp.minimum(his[s], r0 + CHUNK) - base

        def row_body(j, acc):
            return tuple(acc[h] + buf[j, pl.ds(h * LANES, LANES)]
                         for h in range(VPS))

        return lax.fori_loop(jlo, jhi, row_body, acc)

    def out_ref(s):
        return part_hbm.at[pl.ds((g * NSPAN + s) * 1024 + c0, STRIP)]

    zeros = tuple(jnp.zeros((LANES,), jnp.float32) for _ in range(VPS))
    issue(0, 0, bufP0, semP0)
    for s in range(NSPAN):
        bufP, semP = (bufP0, semP0) if s % 2 == 0 else (bufP1, semP1)
        issue(s, 1, bufA, semA)
        issue(s, 2, bufB, semB)
        if s + 1 < NSPAN:
            nbufP, nsemP = (bufP1, semP1) if s % 2 == 0 else (bufP0, semP0)
            issue(s + 1, 0, nbufP, nsemP)
        drain(s, 0, bufP, semP)
        acc = acc_chunk(s, 0, bufP, zeros)

        def make_trip(s):
            def trip_body(k3, acc):
                a = 1 + 3 * k3
                issue(s, a + 2, bufC, semC)
                drain(s, a, bufA, semA)
                acc = acc_chunk(s, a, bufA, acc)
                issue(s, a + 3, bufA, semA)
                drain(s, a + 1, bufB, semB)
                acc = acc_chunk(s, a + 1, bufB, acc)
                issue(s, a + 4, bufB, semB)
                drain(s, a + 2, bufC, semC)
                return acc_chunk(s, a + 2, bufC, acc)
            return trip_body

        acc = lax.fori_loop(0, (ms[s] + 1) // 3, make_trip(s), acc)

        ov, semO = (outv0, semO0) if s % 2 == 0 else (outv1, semO1)
        if s >= 2:
            pltpu.make_async_copy(ov, out_ref(s - 2), semO).wait()
        nv = jnp.full((LANES,), ns[s], jnp.int32).astype(jnp.float32)
        for h in range(VPS):
            ov[pl.ds(h * LANES, LANES)] = acc[h] / nv
        pltpu.async_copy(ov, out_ref(s), semO)

    pltpu.make_async_copy(outv0, out_ref(NSPAN - 2), semO0).wait()
    pltpu.make_async_copy(outv1, out_ref(NSPAN - 1), semO1).wait()


def kernel(sequence_output, entity_positions):
    B, S, H = sequence_output.shape
    pos16 = entity_positions.reshape(B * 4).astype(jnp.int32)

    mesh = plsc.VectorSubcoreMesh(
        core_axis_name="c", subcore_axis_name="s",
        num_cores=NC, num_subcores=NS)
    fn = pl.kernel(
        functools.partial(_span_sum_body, S),
        out_type=jax.ShapeDtypeStruct((NGROUP * NSPAN * H,), jnp.float32),
        mesh=mesh,
        compiler_params=pltpu.CompilerParams(needs_layout_passes=False),
        scratch_types=[
            pltpu.VMEM((16,), jnp.int32),
            pltpu.VMEM((CHUNK, STRIP), jnp.float32),
            pltpu.VMEM((CHUNK, STRIP), jnp.float32),
            pltpu.VMEM((CHUNK, STRIP), jnp.float32),
            pltpu.VMEM((CHUNK, STRIP), jnp.float32),
            pltpu.VMEM((CHUNK, STRIP), jnp.float32),
            pltpu.VMEM((STRIP,), jnp.float32),
            pltpu.VMEM((STRIP,), jnp.float32),
            pltpu.SemaphoreType.DMA,
            pltpu.SemaphoreType.DMA,
            pltpu.SemaphoreType.DMA,
            pltpu.SemaphoreType.DMA,
            pltpu.SemaphoreType.DMA,
            pltpu.SemaphoreType.DMA,
            pltpu.SemaphoreType.DMA,
        ],
    )
    partials = fn(sequence_output, pos16)
    means = partials.reshape(NGROUP, NSPAN, H).sum(axis=0)
    return means[0::2], means[1::2]
